# Initial kernel scaffold; baseline (speedup 1.0000x reference)
import jax
import jax.numpy as jnp
from jax.experimental import pallas as pl


def kernel(z, codebook):
    raise NotImplementedError("write your pallas kernel here")


# speedup vs baseline: 1.3263x; 1.3263x over previous
import jax
import jax.numpy as jnp
from jax.experimental import pallas as pl
from jax.experimental.pallas import tpu as pltpu

_K = 1024
_D = 128
_BETA = 0.25
_TB = 512


def _vq_block_kernel(z_ref, cb_ref, dist_ref, codes_ref, zqt_ref, acc_ref):
    zb = z_ref[0]
    cb = cb_ref[...]

    cross = jax.lax.dot_general(
        zb, cb, (((0,), (1,)), ((), ())),
        preferred_element_type=jnp.float32,
        precision=jax.lax.Precision.DEFAULT)
    z2 = jnp.sum(zb * zb, axis=0)
    c2 = jnp.sum(cb * cb, axis=1)
    dist = z2[:, None] - 2.0 * cross + c2[None, :]
    dist_ref[...] = dist

    min_d = jnp.min(dist, axis=1, keepdims=True)
    iota = jax.lax.broadcasted_iota(jnp.int32, dist.shape, 1)
    is_min = dist == min_d
    codes = jnp.min(jnp.where(is_min, iota, _K), axis=1)
    codes_ref[0, 0] = codes

    onehot = (iota == codes[:, None]).astype(jnp.float32)
    zqt_ref[0] = jax.lax.dot_general(
        cb, onehot, (((0,), (1,)), ((), ())),
        preferred_element_type=jnp.float32,
        precision=jax.lax.Precision.HIGHEST)

    @pl.when(pl.program_id(0) == 0)
    def _init():
        acc_ref[...] = jnp.zeros_like(acc_ref)

    acc_ref[...] += jnp.sum(min_d).reshape(1, 1)


def kernel(z, codebook):
    B, D, T = z.shape
    K = codebook.shape[0]
    n_blocks = (B * T) // _TB
    t_per_b = T // _TB

    grid = (n_blocks,)
    dist_flat, codes_blk, zqt, acc = pl.pallas_call(
        _vq_block_kernel,
        grid=grid,
        in_specs=[
            pl.BlockSpec((1, D, _TB), lambda i: (i // t_per_b, 0, i % t_per_b)),
            pl.BlockSpec((K, D), lambda i: (0, 0)),
        ],
        out_specs=[
            pl.BlockSpec((_TB, K), lambda i: (i, 0)),
            pl.BlockSpec((1, 1, _TB), lambda i: (i, 0, 0)),
            pl.BlockSpec((1, D, _TB), lambda i: (i // t_per_b, 0, i % t_per_b)),
            pl.BlockSpec((1, 1), lambda i: (0, 0)),
        ],
        out_shape=[
            jax.ShapeDtypeStruct((B * T, K), jnp.float32),
            jax.ShapeDtypeStruct((n_blocks, 1, _TB), jnp.int32),
            jax.ShapeDtypeStruct((B, D, T), jnp.float32),
            jax.ShapeDtypeStruct((1, 1), jnp.float32),
        ],
        compiler_params=pltpu.CompilerParams(
            dimension_semantics=("arbitrary",)),
    )(z, codebook)

    codes = codes_blk.reshape(B, T)
    loss = acc[0, 0] * (1.0 + _BETA) / (B * T * D)
    dist = dist_flat.reshape(B, T, K)
    return (zqt, codes, loss, dist)


# speedup vs baseline: 2.5101x; 1.8926x over previous
import jax
import jax.numpy as jnp
from jax.experimental import pallas as pl
from jax.experimental.pallas import tpu as pltpu

_K = 1024
_D = 128
_BETA = 0.25
_TB = 512


def _vq_block_kernel(z_ref, cb_ref, dist_ref, codes_ref, zqt_ref, acc_ref):
    zb = z_ref[0]
    cb = cb_ref[...]

    cross = jax.lax.dot_general(
        zb, cb, (((0,), (1,)), ((), ())),
        preferred_element_type=jnp.float32,
        precision=jax.lax.Precision.DEFAULT)
    z2 = jnp.sum(zb * zb, axis=0)
    c2 = jnp.sum(cb * cb, axis=1)
    dist = z2[:, None] - 2.0 * cross + c2[None, :]
    dist_ref[...] = dist

    min_d = jnp.min(dist, axis=1, keepdims=True)
    iota = jax.lax.broadcasted_iota(jnp.int32, dist.shape, 1)
    is_min = dist == min_d
    codes = jnp.min(jnp.where(is_min, iota, _K), axis=1)
    codes_ref[0, 0] = codes

    onehot = (iota == codes[:, None]).astype(jnp.bfloat16)
    zqt_ref[0] = jax.lax.dot_general(
        cb.astype(jnp.bfloat16), onehot, (((0,), (1,)), ((), ())),
        preferred_element_type=jnp.float32,
        precision=jax.lax.Precision.DEFAULT)

    @pl.when(pl.program_id(0) == 0)
    def _init():
        acc_ref[...] = jnp.zeros_like(acc_ref)

    acc_ref[...] += jnp.sum(min_d).reshape(1, 1)


def kernel(z, codebook):
    B, D, T = z.shape
    K = codebook.shape[0]
    n_blocks = (B * T) // _TB
    t_per_b = T // _TB

    grid = (n_blocks,)
    dist_flat, codes_blk, zqt, acc = pl.pallas_call(
        _vq_block_kernel,
        grid=grid,
        in_specs=[
            pl.BlockSpec((1, D, _TB), lambda i: (i // t_per_b, 0, i % t_per_b)),
            pl.BlockSpec((K, D), lambda i: (0, 0)),
        ],
        out_specs=[
            pl.BlockSpec((_TB, K), lambda i: (i, 0)),
            pl.BlockSpec((1, 1, _TB), lambda i: (i, 0, 0)),
            pl.BlockSpec((1, D, _TB), lambda i: (i // t_per_b, 0, i % t_per_b)),
            pl.BlockSpec((1, 1), lambda i: (0, 0)),
        ],
        out_shape=[
            jax.ShapeDtypeStruct((B * T, K), jnp.float32),
            jax.ShapeDtypeStruct((n_blocks, 1, _TB), jnp.int32),
            jax.ShapeDtypeStruct((B, D, T), jnp.float32),
            jax.ShapeDtypeStruct((1, 1), jnp.float32),
        ],
        compiler_params=pltpu.CompilerParams(
            dimension_semantics=("arbitrary",)),
    )(z, codebook)

    codes = codes_blk.reshape(B, T)
    loss = acc[0, 0] * (1.0 + _BETA) / (B * T * D)
    dist = dist_flat.reshape(B, T, K)
    return (zqt, codes, loss, dist)


# speedup vs baseline: 3.2211x; 1.2833x over previous
import jax
import jax.numpy as jnp
from jax.experimental import pallas as pl
from jax.experimental.pallas import tpu as pltpu

_K = 1024
_D = 128
_BETA = 0.25
_TB = 1024


def _vq_block_kernel(z_ref, cb_ref, dist_ref, codes_ref, zqt_ref, acc_ref,
                     c2_ref, cbh_ref):
    @pl.when(pl.program_id(0) == 0)
    def _init():
        cbf = cb_ref[...]
        c2_ref[...] = jnp.sum(cbf * cbf, axis=1).reshape(1, _K)
        cbh_ref[...] = cbf.astype(jnp.bfloat16)
        acc_ref[...] = jnp.zeros_like(acc_ref)

    zb = z_ref[0]
    cb = cb_ref[...]

    cross = jax.lax.dot_general(
        zb, cb, (((0,), (1,)), ((), ())),
        preferred_element_type=jnp.float32,
        precision=jax.lax.Precision.DEFAULT)
    z2 = jnp.sum(zb * zb, axis=0)
    dist = z2[:, None] - 2.0 * cross + c2_ref[...]
    dist_ref[...] = dist

    min_d = jnp.min(dist, axis=1, keepdims=True)
    iota = jax.lax.broadcasted_iota(jnp.int32, dist.shape, 1)
    codes = jnp.min(jnp.where(dist == min_d, iota, _K), axis=1)
    codes_ref[0, 0] = codes

    onehot = (iota == codes[:, None]).astype(jnp.bfloat16)
    zqt_ref[0] = jax.lax.dot_general(
        cbh_ref[...], onehot, (((0,), (1,)), ((), ())),
        preferred_element_type=jnp.float32,
        precision=jax.lax.Precision.DEFAULT)

    acc_ref[...] += jnp.sum(min_d).reshape(1, 1)


def kernel(z, codebook):
    B, D, T = z.shape
    K = codebook.shape[0]
    n_blocks = (B * T) // _TB
    t_per_b = T // _TB

    grid = (n_blocks,)
    dist_flat, codes_blk, zqt, acc = pl.pallas_call(
        _vq_block_kernel,
        grid=grid,
        in_specs=[
            pl.BlockSpec((1, D, _TB), lambda i: (i // t_per_b, 0, i % t_per_b)),
            pl.BlockSpec((K, D), lambda i: (0, 0)),
        ],
        out_specs=[
            pl.BlockSpec((_TB, K), lambda i: (i, 0)),
            pl.BlockSpec((1, 1, _TB), lambda i: (i, 0, 0)),
            pl.BlockSpec((1, D, _TB), lambda i: (i // t_per_b, 0, i % t_per_b)),
            pl.BlockSpec((1, 1), lambda i: (0, 0)),
        ],
        out_shape=[
            jax.ShapeDtypeStruct((B * T, K), jnp.float32),
            jax.ShapeDtypeStruct((n_blocks, 1, _TB), jnp.int32),
            jax.ShapeDtypeStruct((B, D, T), jnp.float32),
            jax.ShapeDtypeStruct((1, 1), jnp.float32),
        ],
        scratch_shapes=[
            pltpu.VMEM((1, K), jnp.float32),
            pltpu.VMEM((K, D), jnp.bfloat16),
        ],
        compiler_params=pltpu.CompilerParams(
            dimension_semantics=("arbitrary",)),
    )(z, codebook)

    codes = codes_blk.reshape(B, T)
    loss = acc[0, 0] * (1.0 + _BETA) / (B * T * D)
    dist = dist_flat.reshape(B, T, K)
    return (zqt, codes, loss, dist)
